# baseline (speedup 1.0000x reference)
import functools

import jax
import jax.numpy as jnp
from jax import lax
from jax.experimental import pallas as pl
from jax.experimental.pallas import tpu as pltpu
from jax.experimental.pallas import tpu_sc as plsc

_NUM_CODES = 8192
_DIM = 32
_EPS = 1e-6
_BETA = 0.25
_TB = 256
_TBR = 1024


_STRIP = 2048


def _argmin_body(zb_ref, a2_ref, embt2_ref, b2_ref, idx_ref):
    zb = zb_ref[...]
    embt2 = embt2_ref[...]
    ab2 = lax.dot_general(zb, embt2, (((1,), (0,)), ((), ())),
                          preferred_element_type=jnp.float32)
    a2 = a2_ref[...]
    b2 = b2_ref[...]
    accv = None
    acci = None
    lane = lax.broadcasted_iota(jnp.int32, (_TB, 128), 1)
    for s in range(_NUM_CODES // _STRIP):
        mv = None
        for c in range(_STRIP // 128):
            lo = s * _STRIP + c * 128
            d = (a2 - ab2[:, lo:lo + 128]) + b2[:, lo:lo + 128]
            if c == 0:
                mv = d
                mc = jnp.zeros((_TB, 128), jnp.int32)
            else:
                lt = d < mv
                mv = jnp.where(lt, d, mv)
                mc = jnp.where(lt, c, mc)
        m = jnp.min(mv, axis=1, keepdims=True)
        mi_full = mc * 128 + lane + (s * _STRIP)
        mi = jnp.min(jnp.where(mv == m, mi_full, _NUM_CODES), axis=1,
                     keepdims=True)
        if s == 0:
            accv = m.astype(jnp.bfloat16).astype(jnp.float32)
            acci = mi
        else:
            take = m < accv
            accv = jnp.where(take, m, accv).astype(jnp.bfloat16).astype(jnp.float32)
            acci = jnp.where(take, mi, acci)
    idx_ref[...] = acci


def _argmin_call(zb, a2, embt, b2, interpret=False):
    n = zb.shape[0]
    return pl.pallas_call(
        _argmin_body,
        grid=(n // _TB,),
        in_specs=[
            pl.BlockSpec((_TB, _DIM), lambda i: (i, 0)),
            pl.BlockSpec((_TB, 1), lambda i: (i, 0)),
            pl.BlockSpec((_DIM, _NUM_CODES), lambda i: (0, 0)),
            pl.BlockSpec((1, _NUM_CODES), lambda i: (0, 0)),
        ],
        out_specs=pl.BlockSpec((_TB, 1), lambda i: (i, 0)),
        out_shape=jax.ShapeDtypeStruct((n, 1), jnp.int32),
        interpret=interpret,
    )(zb, a2, embt, b2)


def _sc_gather(embedding, idx_flat):
    n = idx_flat.shape[0]
    info = plsc.get_sparse_core_info()
    nc, ns = info.num_cores, info.num_subcores
    nw = nc * ns
    b_per_w = n // nw
    mesh = plsc.VectorSubcoreMesh(core_axis_name="c", subcore_axis_name="s")

    @functools.partial(
        pl.kernel,
        mesh=mesh,
        compiler_params=pltpu.CompilerParams(use_tc_tiling_on_sc=False),
        out_type=jax.ShapeDtypeStruct((n, _DIM), jnp.float32),
        scratch_types=[
            pltpu.VMEM((b_per_w,), jnp.int32),
            pltpu.VMEM((b_per_w, _DIM), jnp.float32),
            pltpu.SemaphoreType.DMA,
        ],
    )
    def gather_k(table_hbm, idx_hbm, out_hbm, idx_v, rows_v, sem):
        wid = lax.axis_index("s") * nc + lax.axis_index("c")
        base = wid * b_per_w
        pltpu.sync_copy(idx_hbm.at[pl.ds(base, b_per_w)], idx_v)
        pltpu.async_copy(table_hbm.at[idx_v], rows_v, sem).wait()
        pltpu.sync_copy(rows_v, out_hbm.at[pl.ds(base, b_per_w)])

    return gather_k(embedding, idx_flat)


def _rotation_body(z_ref, q_ref, zq_ref, loss_ref):
    z = z_ref[...]
    q = q_ref[...]
    zn = jnp.sqrt(jnp.sum(z * z, axis=1, keepdims=True))
    qn = jnp.sqrt(jnp.sum(q * q, axis=1, keepdims=True))
    z_hat = z / (zn + _EPS)
    q_hat = q / (qn + _EPS)
    v = z_hat - q_hat
    v = v / (jnp.sqrt(jnp.sum(v * v, axis=1, keepdims=True)) + _EPS)
    rz = z - 2.0 * jnp.sum(v * z, axis=1, keepdims=True) * v
    s = qn / (zn + _EPS)
    zq_ref[...] = s * rz

    @pl.when(pl.program_id(0) == 0)
    def _():
        loss_ref[...] = jnp.zeros((1, 1), jnp.float32)

    d = z - q
    loss_ref[...] += jnp.sum(d * d).reshape(1, 1)


def _rotation_call(z_flat, q_flat, interpret=False):
    n = z_flat.shape[0]
    return pl.pallas_call(
        _rotation_body,
        grid=(n // _TBR,),
        in_specs=[
            pl.BlockSpec((_TBR, _DIM), lambda i: (i, 0)),
            pl.BlockSpec((_TBR, _DIM), lambda i: (i, 0)),
        ],
        out_specs=[
            pl.BlockSpec((_TBR, _DIM), lambda i: (i, 0)),
            pl.BlockSpec((1, 1), lambda i: (0, 0)),
        ],
        out_shape=[
            jax.ShapeDtypeStruct((n, _DIM), jnp.float32),
            jax.ShapeDtypeStruct((1, 1), jnp.float32),
        ],
        interpret=interpret,
    )(z_flat, q_flat)


_CHUNKS = 2


def kernel(z_e, embedding):
    b, d, h, w = z_e.shape
    n = b * h * w
    z_flat = jnp.transpose(z_e, (0, 2, 3, 1)).reshape(n, d)
    embt2 = embedding.T * 2.0
    zb = z_flat.astype(jnp.bfloat16).astype(jnp.float32)
    a2 = jnp.sum(z_flat ** 2, axis=1, keepdims=True)
    b2 = jnp.sum(embedding ** 2, axis=1).reshape(1, _NUM_CODES)

    nc = n // _CHUNKS
    idxs, qs = [], []
    for k in range(_CHUNKS):
        sl = slice(k * nc, (k + 1) * nc)
        idx2 = _argmin_call(zb[sl], a2[sl], embt2, b2)
        idx_flat = idx2.reshape(nc)
        idxs.append(idx_flat)
        qs.append(_sc_gather(embedding, idx_flat))

    q_tildes, losses = [], []
    for k in range(_CHUNKS):
        sl = slice(k * nc, (k + 1) * nc)
        q_tilde, loss_sum = _rotation_call(z_flat[sl], qs[k])
        q_tildes.append(q_tilde)
        losses.append(loss_sum[0, 0])

    commit_loss = sum(losses) * (_BETA / (n * d))
    q_all = jnp.concatenate(q_tildes, axis=0)
    z_q = jnp.transpose(q_all.reshape(b, h, w, d), (0, 3, 1, 2))
    indices_out = jnp.concatenate(idxs, axis=0).reshape(b, h, w)
    return (z_q, indices_out, commit_loss)


# speedup vs baseline: 1.1601x; 1.1601x over previous
import functools

import jax
import jax.numpy as jnp
from jax import lax
from jax.experimental import pallas as pl
from jax.experimental.pallas import tpu as pltpu
from jax.experimental.pallas import tpu_sc as plsc

_NUM_CODES = 8192
_DIM = 32
_EPS = 1e-6
_BETA = 0.25
_TB = 256
_TBR = 2048


_STRIP = 2048


def _argmin_body(zb_ref, a2_ref, embt2_ref, b2_ref, idx_ref):
    zb = zb_ref[...]
    embt2 = embt2_ref[...]
    ab2 = lax.dot_general(zb, embt2, (((1,), (0,)), ((), ())),
                          preferred_element_type=jnp.float32)
    a2 = a2_ref[...]
    b2 = b2_ref[...]
    accv = None
    acci = None
    lane = lax.broadcasted_iota(jnp.int32, (_TB, 128), 1)
    for s in range(_NUM_CODES // _STRIP):
        mv = None
        for c in range(_STRIP // 128):
            lo = s * _STRIP + c * 128
            d = (a2 - ab2[:, lo:lo + 128]) + b2[:, lo:lo + 128]
            if c == 0:
                mv = d
                mc = jnp.zeros((_TB, 128), jnp.int32)
            else:
                lt = d < mv
                mv = jnp.where(lt, d, mv)
                mc = jnp.where(lt, c, mc)
        m = jnp.min(mv, axis=1, keepdims=True)
        mi_full = mc * 128 + lane + (s * _STRIP)
        mi = jnp.min(jnp.where(mv == m, mi_full, _NUM_CODES), axis=1,
                     keepdims=True)
        if s == 0:
            accv = m.astype(jnp.bfloat16).astype(jnp.float32)
            acci = mi
        else:
            take = m < accv
            accv = jnp.where(take, m, accv).astype(jnp.bfloat16).astype(jnp.float32)
            acci = jnp.where(take, mi, acci)
    idx_ref[...] = acci


def _argmin_call(zb, a2, embt, b2, interpret=False):
    n = zb.shape[0]
    return pl.pallas_call(
        _argmin_body,
        grid=(n // _TB,),
        in_specs=[
            pl.BlockSpec((_TB, _DIM), lambda i: (i, 0)),
            pl.BlockSpec((_TB, 1), lambda i: (i, 0)),
            pl.BlockSpec((_DIM, _NUM_CODES), lambda i: (0, 0)),
            pl.BlockSpec((1, _NUM_CODES), lambda i: (0, 0)),
        ],
        out_specs=pl.BlockSpec((_TB, 1), lambda i: (i, 0)),
        out_shape=jax.ShapeDtypeStruct((n, 1), jnp.int32),
        interpret=interpret,
    )(zb, a2, embt, b2)


def _sc_gather(embedding, idx_flat):
    n = idx_flat.shape[0]
    info = plsc.get_sparse_core_info()
    nc, ns = info.num_cores, info.num_subcores
    nw = nc * ns
    b_per_w = n // nw
    mesh = plsc.VectorSubcoreMesh(core_axis_name="c", subcore_axis_name="s")

    @functools.partial(
        pl.kernel,
        mesh=mesh,
        compiler_params=pltpu.CompilerParams(use_tc_tiling_on_sc=False),
        out_type=jax.ShapeDtypeStruct((n, _DIM), jnp.float32),
        scratch_types=[
            pltpu.VMEM((b_per_w,), jnp.int32),
            pltpu.VMEM((b_per_w, _DIM), jnp.float32),
            pltpu.SemaphoreType.DMA,
        ],
    )
    def gather_k(table_hbm, idx_hbm, out_hbm, idx_v, rows_v, sem):
        wid = lax.axis_index("s") * nc + lax.axis_index("c")
        base = wid * b_per_w
        pltpu.sync_copy(idx_hbm.at[pl.ds(base, b_per_w)], idx_v)
        pltpu.async_copy(table_hbm.at[idx_v], rows_v, sem).wait()
        pltpu.sync_copy(rows_v, out_hbm.at[pl.ds(base, b_per_w)])

    return gather_k(embedding, idx_flat)


def _rotation_body(z_ref, q_ref, zq_ref, loss_ref):
    z = z_ref[...]
    q = q_ref[...]
    zn = jnp.sqrt(jnp.sum(z * z, axis=1, keepdims=True))
    qn = jnp.sqrt(jnp.sum(q * q, axis=1, keepdims=True))
    z_hat = z / (zn + _EPS)
    q_hat = q / (qn + _EPS)
    v = z_hat - q_hat
    v = v / (jnp.sqrt(jnp.sum(v * v, axis=1, keepdims=True)) + _EPS)
    rz = z - 2.0 * jnp.sum(v * z, axis=1, keepdims=True) * v
    s = qn / (zn + _EPS)
    zq_ref[...] = s * rz

    @pl.when(pl.program_id(0) == 0)
    def _():
        loss_ref[...] = jnp.zeros((1, 1), jnp.float32)

    d = z - q
    loss_ref[...] += jnp.sum(d * d).reshape(1, 1)


def _rotation_call(z_flat, q_flat, interpret=False):
    n = z_flat.shape[0]
    return pl.pallas_call(
        _rotation_body,
        grid=(n // _TBR,),
        in_specs=[
            pl.BlockSpec((_TBR, _DIM), lambda i: (i, 0)),
            pl.BlockSpec((_TBR, _DIM), lambda i: (i, 0)),
        ],
        out_specs=[
            pl.BlockSpec((_TBR, _DIM), lambda i: (i, 0)),
            pl.BlockSpec((1, 1), lambda i: (0, 0)),
        ],
        out_shape=[
            jax.ShapeDtypeStruct((n, _DIM), jnp.float32),
            jax.ShapeDtypeStruct((1, 1), jnp.float32),
        ],
        interpret=interpret,
    )(z_flat, q_flat)


def kernel(z_e, embedding):
    b, d, h, w = z_e.shape
    n = b * h * w
    z_flat = jnp.transpose(z_e, (0, 2, 3, 1)).reshape(n, d)
    embt2 = embedding.T * 2.0
    zb = z_flat.astype(jnp.bfloat16)
    a2 = jnp.sum(z_flat ** 2, axis=1, keepdims=True)
    b2 = jnp.sum(embedding ** 2, axis=1).reshape(1, _NUM_CODES)

    idx2 = _argmin_call(zb, a2, embt2, b2)
    idx_flat = idx2.reshape(n)
    q_flat = _sc_gather(embedding, idx_flat)
    q_tilde, loss_sum = _rotation_call(z_flat, q_flat)

    commit_loss = loss_sum[0, 0] * (_BETA / (n * d))
    z_q = jnp.transpose(q_tilde.reshape(b, h, w, d), (0, 3, 1, 2))
    indices_out = idx_flat.reshape(b, h, w)
    return (z_q, indices_out, commit_loss)


# speedup vs baseline: 1.1970x; 1.0319x over previous
import functools

import jax
import jax.numpy as jnp
from jax import lax
from jax.experimental import pallas as pl
from jax.experimental.pallas import tpu as pltpu
from jax.experimental.pallas import tpu_sc as plsc

_NUM_CODES = 8192
_DIM = 32
_EPS = 1e-6
_BETA = 0.25
_TB = 512
_TBR = 8192


_STRIP = 2048


def _argmin_body(zb_ref, a2_ref, embt2_ref, b2_ref, idx_ref):
    zb = zb_ref[...]
    embt2 = embt2_ref[...]
    ab2 = lax.dot_general(zb, embt2, (((1,), (0,)), ((), ())),
                          preferred_element_type=jnp.float32)
    a2 = a2_ref[...]
    b2 = b2_ref[...]
    accv = None
    acci = None
    lane = lax.broadcasted_iota(jnp.int32, (_TB, 128), 1)
    for s in range(_NUM_CODES // _STRIP):
        mv = None
        for c in range(_STRIP // 128):
            lo = s * _STRIP + c * 128
            d = (a2 - ab2[:, lo:lo + 128]) + b2[:, lo:lo + 128]
            if c == 0:
                mv = d
                mc = jnp.zeros((_TB, 128), jnp.int32)
            else:
                lt = d < mv
                mv = jnp.where(lt, d, mv)
                mc = jnp.where(lt, c, mc)
        m = jnp.min(mv, axis=1, keepdims=True)
        mi_full = mc * 128 + lane + (s * _STRIP)
        mi = jnp.min(jnp.where(mv == m, mi_full, _NUM_CODES), axis=1,
                     keepdims=True)
        if s == 0:
            accv = m.astype(jnp.bfloat16).astype(jnp.float32)
            acci = mi
        else:
            take = m < accv
            accv = jnp.where(take, m, accv).astype(jnp.bfloat16).astype(jnp.float32)
            acci = jnp.where(take, mi, acci)
    idx_ref[...] = acci


def _argmin_call(zb, a2, embt, b2, interpret=False):
    n = zb.shape[0]
    return pl.pallas_call(
        _argmin_body,
        grid=(n // _TB,),
        in_specs=[
            pl.BlockSpec((_TB, _DIM), lambda i: (i, 0)),
            pl.BlockSpec((_TB, 1), lambda i: (i, 0)),
            pl.BlockSpec((_DIM, _NUM_CODES), lambda i: (0, 0)),
            pl.BlockSpec((1, _NUM_CODES), lambda i: (0, 0)),
        ],
        out_specs=pl.BlockSpec((_TB, 1), lambda i: (i, 0)),
        out_shape=jax.ShapeDtypeStruct((n, 1), jnp.int32),
        interpret=interpret,
    )(zb, a2, embt, b2)


def _sc_gather(embedding, idx_flat):
    n = idx_flat.shape[0]
    info = plsc.get_sparse_core_info()
    nc, ns = info.num_cores, info.num_subcores
    nw = nc * ns
    b_per_w = n // nw
    mesh = plsc.VectorSubcoreMesh(core_axis_name="c", subcore_axis_name="s")

    @functools.partial(
        pl.kernel,
        mesh=mesh,
        compiler_params=pltpu.CompilerParams(use_tc_tiling_on_sc=False),
        out_type=jax.ShapeDtypeStruct((n, _DIM), jnp.float32),
        scratch_types=[
            pltpu.VMEM((b_per_w,), jnp.int32),
            pltpu.VMEM((b_per_w, _DIM), jnp.float32),
            pltpu.SemaphoreType.DMA,
        ],
    )
    def gather_k(table_hbm, idx_hbm, out_hbm, idx_v, rows_v, sem):
        wid = lax.axis_index("s") * nc + lax.axis_index("c")
        base = wid * b_per_w
        pltpu.sync_copy(idx_hbm.at[pl.ds(base, b_per_w)], idx_v)
        pltpu.async_copy(table_hbm.at[idx_v], rows_v, sem).wait()
        pltpu.sync_copy(rows_v, out_hbm.at[pl.ds(base, b_per_w)])

    return gather_k(embedding, idx_flat)


def _rotation_body(z_ref, q_ref, zq_ref, loss_ref):
    z = z_ref[...]
    q = q_ref[...]
    zn = jnp.sqrt(jnp.sum(z * z, axis=1, keepdims=True))
    qn = jnp.sqrt(jnp.sum(q * q, axis=1, keepdims=True))
    z_hat = z / (zn + _EPS)
    q_hat = q / (qn + _EPS)
    v = z_hat - q_hat
    v = v / (jnp.sqrt(jnp.sum(v * v, axis=1, keepdims=True)) + _EPS)
    rz = z - 2.0 * jnp.sum(v * z, axis=1, keepdims=True) * v
    s = qn / (zn + _EPS)
    zq_ref[...] = s * rz

    @pl.when(pl.program_id(0) == 0)
    def _():
        loss_ref[...] = jnp.zeros((1, 1), jnp.float32)

    d = z - q
    loss_ref[...] += jnp.sum(d * d).reshape(1, 1)


def _rotation_call(z_flat, q_flat, interpret=False):
    n = z_flat.shape[0]
    return pl.pallas_call(
        _rotation_body,
        grid=(n // _TBR,),
        in_specs=[
            pl.BlockSpec((_TBR, _DIM), lambda i: (i, 0)),
            pl.BlockSpec((_TBR, _DIM), lambda i: (i, 0)),
        ],
        out_specs=[
            pl.BlockSpec((_TBR, _DIM), lambda i: (i, 0)),
            pl.BlockSpec((1, 1), lambda i: (0, 0)),
        ],
        out_shape=[
            jax.ShapeDtypeStruct((n, _DIM), jnp.float32),
            jax.ShapeDtypeStruct((1, 1), jnp.float32),
        ],
        interpret=interpret,
    )(z_flat, q_flat)


def kernel(z_e, embedding):
    b, d, h, w = z_e.shape
    n = b * h * w
    z_flat = jnp.transpose(z_e, (0, 2, 3, 1)).reshape(n, d)
    embt2 = embedding.T * 2.0
    zb = z_flat.astype(jnp.bfloat16)
    a2 = jnp.sum(z_flat ** 2, axis=1, keepdims=True)
    b2 = jnp.sum(embedding ** 2, axis=1).reshape(1, _NUM_CODES)

    idx2 = _argmin_call(zb, a2, embt2, b2)
    idx_flat = idx2.reshape(n)
    q_flat = _sc_gather(embedding, idx_flat)
    q_tilde, loss_sum = _rotation_call(z_flat, q_flat)

    commit_loss = loss_sum[0, 0] * (_BETA / (n * d))
    z_q = jnp.transpose(q_tilde.reshape(b, h, w, d), (0, 3, 1, 2))
    indices_out = idx_flat.reshape(b, h, w)
    return (z_q, indices_out, commit_loss)


# speedup vs baseline: 1.2269x; 1.0249x over previous
import functools

import jax
import jax.numpy as jnp
from jax import lax
from jax.experimental import pallas as pl
from jax.experimental.pallas import tpu as pltpu
from jax.experimental.pallas import tpu_sc as plsc

_NUM_CODES = 8192
_DIM = 32
_EPS = 1e-6
_BETA = 0.25
_TB = 1024
_TBR = 8192


_STRIP = 2048


def _argmin_body(zb_ref, a2_ref, embt2_ref, b2_ref, idx_ref):
    zb = zb_ref[...]
    embt2 = embt2_ref[...]
    ab2 = lax.dot_general(zb, embt2, (((1,), (0,)), ((), ())),
                          preferred_element_type=jnp.float32)
    a2 = a2_ref[...]
    b2 = b2_ref[...]
    accv = None
    acci = None
    lane = lax.broadcasted_iota(jnp.int32, (_TB, 128), 1)
    for s in range(_NUM_CODES // _STRIP):
        mv = None
        for c in range(_STRIP // 128):
            lo = s * _STRIP + c * 128
            d = (a2 - ab2[:, lo:lo + 128]) + b2[:, lo:lo + 128]
            if c == 0:
                mv = d
                mc = jnp.zeros((_TB, 128), jnp.int32)
            else:
                lt = d < mv
                mv = jnp.where(lt, d, mv)
                mc = jnp.where(lt, c, mc)
        m = jnp.min(mv, axis=1, keepdims=True)
        mi_full = mc * 128 + lane + (s * _STRIP)
        mi = jnp.min(jnp.where(mv == m, mi_full, _NUM_CODES), axis=1,
                     keepdims=True)
        if s == 0:
            accv = m.astype(jnp.bfloat16).astype(jnp.float32)
            acci = mi
        else:
            take = m < accv
            accv = jnp.where(take, m, accv).astype(jnp.bfloat16).astype(jnp.float32)
            acci = jnp.where(take, mi, acci)
    idx_ref[...] = acci


def _argmin_call(zb, a2, embt, b2, interpret=False):
    n = zb.shape[0]
    return pl.pallas_call(
        _argmin_body,
        grid=(n // _TB,),
        in_specs=[
            pl.BlockSpec((_TB, _DIM), lambda i: (i, 0)),
            pl.BlockSpec((_TB, 1), lambda i: (i, 0)),
            pl.BlockSpec((_DIM, _NUM_CODES), lambda i: (0, 0)),
            pl.BlockSpec((1, _NUM_CODES), lambda i: (0, 0)),
        ],
        out_specs=pl.BlockSpec((_TB, 1), lambda i: (i, 0)),
        out_shape=jax.ShapeDtypeStruct((n, 1), jnp.int32),
        interpret=interpret,
    )(zb, a2, embt, b2)


def _sc_gather(embedding, idx_flat):
    n = idx_flat.shape[0]
    info = plsc.get_sparse_core_info()
    nc, ns = info.num_cores, info.num_subcores
    nw = nc * ns
    b_per_w = n // nw
    mesh = plsc.VectorSubcoreMesh(core_axis_name="c", subcore_axis_name="s")

    @functools.partial(
        pl.kernel,
        mesh=mesh,
        compiler_params=pltpu.CompilerParams(use_tc_tiling_on_sc=False),
        out_type=jax.ShapeDtypeStruct((n, _DIM), jnp.float32),
        scratch_types=[
            pltpu.VMEM((b_per_w,), jnp.int32),
            pltpu.VMEM((b_per_w, _DIM), jnp.float32),
            pltpu.SemaphoreType.DMA,
        ],
    )
    def gather_k(table_hbm, idx_hbm, out_hbm, idx_v, rows_v, sem):
        wid = lax.axis_index("s") * nc + lax.axis_index("c")
        base = wid * b_per_w
        pltpu.sync_copy(idx_hbm.at[pl.ds(base, b_per_w)], idx_v)
        pltpu.async_copy(table_hbm.at[idx_v], rows_v, sem).wait()
        pltpu.sync_copy(rows_v, out_hbm.at[pl.ds(base, b_per_w)])

    return gather_k(embedding, idx_flat)


def _rotation_body(z_ref, q_ref, zq_ref, loss_ref):
    z = z_ref[...]
    q = q_ref[...]
    zn = jnp.sqrt(jnp.sum(z * z, axis=1, keepdims=True))
    qn = jnp.sqrt(jnp.sum(q * q, axis=1, keepdims=True))
    z_hat = z / (zn + _EPS)
    q_hat = q / (qn + _EPS)
    v = z_hat - q_hat
    v = v / (jnp.sqrt(jnp.sum(v * v, axis=1, keepdims=True)) + _EPS)
    rz = z - 2.0 * jnp.sum(v * z, axis=1, keepdims=True) * v
    s = qn / (zn + _EPS)
    zq_ref[...] = s * rz

    @pl.when(pl.program_id(0) == 0)
    def _():
        loss_ref[...] = jnp.zeros((1, 1), jnp.float32)

    d = z - q
    loss_ref[...] += jnp.sum(d * d).reshape(1, 1)


def _rotation_call(z_flat, q_flat, interpret=False):
    n = z_flat.shape[0]
    return pl.pallas_call(
        _rotation_body,
        grid=(n // _TBR,),
        in_specs=[
            pl.BlockSpec((_TBR, _DIM), lambda i: (i, 0)),
            pl.BlockSpec((_TBR, _DIM), lambda i: (i, 0)),
        ],
        out_specs=[
            pl.BlockSpec((_TBR, _DIM), lambda i: (i, 0)),
            pl.BlockSpec((1, 1), lambda i: (0, 0)),
        ],
        out_shape=[
            jax.ShapeDtypeStruct((n, _DIM), jnp.float32),
            jax.ShapeDtypeStruct((1, 1), jnp.float32),
        ],
        interpret=interpret,
    )(z_flat, q_flat)


def kernel(z_e, embedding):
    b, d, h, w = z_e.shape
    n = b * h * w
    z_flat = jnp.transpose(z_e, (0, 2, 3, 1)).reshape(n, d)
    embt2 = embedding.T * 2.0
    zb = z_flat.astype(jnp.bfloat16)
    a2 = jnp.sum(z_flat ** 2, axis=1, keepdims=True)
    b2 = jnp.sum(embedding ** 2, axis=1).reshape(1, _NUM_CODES)

    idx2 = _argmin_call(zb, a2, embt2, b2)
    idx_flat = idx2.reshape(n)
    q_flat = _sc_gather(embedding, idx_flat)
    q_tilde, loss_sum = _rotation_call(z_flat, q_flat)

    commit_loss = loss_sum[0, 0] * (_BETA / (n * d))
    z_q = jnp.transpose(q_tilde.reshape(b, h, w, d), (0, 3, 1, 2))
    indices_out = idx_flat.reshape(b, h, w)
    return (z_q, indices_out, commit_loss)


# speedup vs baseline: 1.2487x; 1.0178x over previous
import functools

import jax
import jax.numpy as jnp
from jax import lax
from jax.experimental import pallas as pl
from jax.experimental.pallas import tpu as pltpu
from jax.experimental.pallas import tpu_sc as plsc

_NUM_CODES = 8192
_DIM = 32
_EPS = 1e-6
_BETA = 0.25
_TB = 2048
_TBR = 8192


_STRIP = 2048


def _argmin_body(zb_ref, a2_ref, embt2_ref, b2_ref, idx_ref):
    zb = zb_ref[...]
    embt2 = embt2_ref[...]
    ab2 = lax.dot_general(zb, embt2, (((1,), (0,)), ((), ())),
                          preferred_element_type=jnp.float32)
    a2 = a2_ref[...]
    b2 = b2_ref[...]
    accv = None
    acci = None
    lane = lax.broadcasted_iota(jnp.int32, (_TB, 128), 1)
    for s in range(_NUM_CODES // _STRIP):
        mv = None
        for c in range(_STRIP // 128):
            lo = s * _STRIP + c * 128
            d = (a2 - ab2[:, lo:lo + 128]) + b2[:, lo:lo + 128]
            if c == 0:
                mv = d
                mc = jnp.zeros((_TB, 128), jnp.int32)
            else:
                lt = d < mv
                mv = jnp.where(lt, d, mv)
                mc = jnp.where(lt, c, mc)
        m = jnp.min(mv, axis=1, keepdims=True)
        mi_full = mc * 128 + lane + (s * _STRIP)
        mi = jnp.min(jnp.where(mv == m, mi_full, _NUM_CODES), axis=1,
                     keepdims=True)
        if s == 0:
            accv = m.astype(jnp.bfloat16).astype(jnp.float32)
            acci = mi
        else:
            take = m < accv
            accv = jnp.where(take, m, accv).astype(jnp.bfloat16).astype(jnp.float32)
            acci = jnp.where(take, mi, acci)
    idx_ref[...] = acci


def _argmin_call(zb, a2, embt, b2, interpret=False):
    n = zb.shape[0]
    return pl.pallas_call(
        _argmin_body,
        grid=(n // _TB,),
        in_specs=[
            pl.BlockSpec((_TB, _DIM), lambda i: (i, 0)),
            pl.BlockSpec((_TB, 1), lambda i: (i, 0)),
            pl.BlockSpec((_DIM, _NUM_CODES), lambda i: (0, 0)),
            pl.BlockSpec((1, _NUM_CODES), lambda i: (0, 0)),
        ],
        out_specs=pl.BlockSpec((_TB, 1), lambda i: (i, 0)),
        out_shape=jax.ShapeDtypeStruct((n, 1), jnp.int32),
        interpret=interpret,
    )(zb, a2, embt, b2)


def _sc_gather(embedding, idx_flat):
    n = idx_flat.shape[0]
    info = plsc.get_sparse_core_info()
    nc, ns = info.num_cores, info.num_subcores
    nw = nc * ns
    b_per_w = n // nw
    mesh = plsc.VectorSubcoreMesh(core_axis_name="c", subcore_axis_name="s")

    @functools.partial(
        pl.kernel,
        mesh=mesh,
        compiler_params=pltpu.CompilerParams(use_tc_tiling_on_sc=False),
        out_type=jax.ShapeDtypeStruct((n, _DIM), jnp.float32),
        scratch_types=[
            pltpu.VMEM((b_per_w,), jnp.int32),
            pltpu.VMEM((b_per_w, _DIM), jnp.float32),
            pltpu.SemaphoreType.DMA,
        ],
    )
    def gather_k(table_hbm, idx_hbm, out_hbm, idx_v, rows_v, sem):
        wid = lax.axis_index("s") * nc + lax.axis_index("c")
        base = wid * b_per_w
        pltpu.sync_copy(idx_hbm.at[pl.ds(base, b_per_w)], idx_v)
        pltpu.async_copy(table_hbm.at[idx_v], rows_v, sem).wait()
        pltpu.sync_copy(rows_v, out_hbm.at[pl.ds(base, b_per_w)])

    return gather_k(embedding, idx_flat)


def _rotation_body(z_ref, q_ref, zq_ref, loss_ref):
    z = z_ref[...]
    q = q_ref[...]
    zn = jnp.sqrt(jnp.sum(z * z, axis=1, keepdims=True))
    qn = jnp.sqrt(jnp.sum(q * q, axis=1, keepdims=True))
    z_hat = z / (zn + _EPS)
    q_hat = q / (qn + _EPS)
    v = z_hat - q_hat
    v = v / (jnp.sqrt(jnp.sum(v * v, axis=1, keepdims=True)) + _EPS)
    rz = z - 2.0 * jnp.sum(v * z, axis=1, keepdims=True) * v
    s = qn / (zn + _EPS)
    zq_ref[...] = s * rz

    @pl.when(pl.program_id(0) == 0)
    def _():
        loss_ref[...] = jnp.zeros((1, 1), jnp.float32)

    d = z - q
    loss_ref[...] += jnp.sum(d * d).reshape(1, 1)


def _rotation_call(z_flat, q_flat, interpret=False):
    n = z_flat.shape[0]
    return pl.pallas_call(
        _rotation_body,
        grid=(n // _TBR,),
        in_specs=[
            pl.BlockSpec((_TBR, _DIM), lambda i: (i, 0)),
            pl.BlockSpec((_TBR, _DIM), lambda i: (i, 0)),
        ],
        out_specs=[
            pl.BlockSpec((_TBR, _DIM), lambda i: (i, 0)),
            pl.BlockSpec((1, 1), lambda i: (0, 0)),
        ],
        out_shape=[
            jax.ShapeDtypeStruct((n, _DIM), jnp.float32),
            jax.ShapeDtypeStruct((1, 1), jnp.float32),
        ],
        interpret=interpret,
    )(z_flat, q_flat)


def kernel(z_e, embedding):
    b, d, h, w = z_e.shape
    n = b * h * w
    z_flat = jnp.transpose(z_e, (0, 2, 3, 1)).reshape(n, d)
    embt2 = embedding.T * 2.0
    zb = z_flat.astype(jnp.bfloat16)
    a2 = jnp.sum(z_flat ** 2, axis=1, keepdims=True)
    b2 = jnp.sum(embedding ** 2, axis=1).reshape(1, _NUM_CODES)

    idx2 = _argmin_call(zb, a2, embt2, b2)
    idx_flat = idx2.reshape(n)
    q_flat = _sc_gather(embedding, idx_flat)
    q_tilde, loss_sum = _rotation_call(z_flat, q_flat)

    commit_loss = loss_sum[0, 0] * (_BETA / (n * d))
    z_q = jnp.transpose(q_tilde.reshape(b, h, w, d), (0, 3, 1, 2))
    indices_out = idx_flat.reshape(b, h, w)
    return (z_q, indices_out, commit_loss)
